# Initial kernel scaffold; baseline (speedup 1.0000x reference)
import jax
import jax.numpy as jnp
from jax.experimental import pallas as pl


def kernel(inputs, codebook):
    raise NotImplementedError("write your pallas kernel here")


# speedup vs baseline: 6.9791x; 6.9791x over previous
import functools

import jax
import jax.numpy as jnp
from jax.experimental import pallas as pl
from jax.experimental.pallas import tpu as pltpu

S, N, D = 1024, 8, 256
M = S * N
K = 8192
BM = 256


def _vq_block_kernel(x_ref, cb_ref, xsq_ref, csq_ref,
                     loss_ref, q_ref, oh_ref, idx_ref):
    i = pl.program_id(0)
    x = x_ref[...]
    cb = cb_ref[...]
    mm = jax.lax.dot_general(x, cb, (((1,), (1,)), ((), ())),
                             preferred_element_type=jnp.float32)
    d = (xsq_ref[...] + csq_ref[...]) - 2.0 * mm
    dmin = jnp.min(d, axis=1, keepdims=True)
    kio = jax.lax.broadcasted_iota(jnp.int32, d.shape, 1)
    idx = jnp.min(jnp.where(d == dmin, kio, K), axis=1, keepdims=True)
    idx_ref[...] = idx
    oh = (kio == idx).astype(jnp.float32)
    oh_ref[...] = oh
    q = jax.lax.dot_general(oh, cb, (((1,), (0,)), ((), ())),
                            preferred_element_type=jnp.float32)
    q_ref[...] = q

    @pl.when(i == 0)
    def _init():
        loss_ref[...] = jnp.zeros_like(loss_ref)

    loss_ref[...] += jnp.sum((q - x) ** 2).reshape(1, 1)


@jax.jit
def kernel(inputs, codebook):
    flat = inputs.reshape(-1, D)
    xsq = jnp.sum(flat ** 2, axis=1, keepdims=True)
    csq = jnp.sum(codebook ** 2, axis=1)[None, :]

    grid = (M // BM,)
    loss_acc, q, oh, idx = pl.pallas_call(
        _vq_block_kernel,
        grid=grid,
        in_specs=[
            pl.BlockSpec((BM, D), lambda i: (i, 0)),
            pl.BlockSpec((K, D), lambda i: (0, 0)),
            pl.BlockSpec((BM, 1), lambda i: (i, 0)),
            pl.BlockSpec((1, K), lambda i: (0, 0)),
        ],
        out_specs=[
            pl.BlockSpec((1, 1), lambda i: (0, 0)),
            pl.BlockSpec((BM, D), lambda i: (i, 0)),
            pl.BlockSpec((BM, K), lambda i: (i, 0)),
            pl.BlockSpec((BM, 1), lambda i: (i, 0)),
        ],
        out_shape=[
            jax.ShapeDtypeStruct((1, 1), jnp.float32),
            jax.ShapeDtypeStruct((M, D), jnp.float32),
            jax.ShapeDtypeStruct((M, K), jnp.float32),
            jax.ShapeDtypeStruct((M, 1), jnp.int32),
        ],
    )(flat, codebook, xsq, csq)

    loss = loss_acc[0, 0] * (0.25 / (M * D))
    quantized_st = q.reshape(S, N, D)
    encodings_flat = oh.reshape(S, N, K)
    return (loss, quantized_st, encodings_flat, idx)


# speedup vs baseline: 10.1160x; 1.4495x over previous
import functools

import jax
import jax.numpy as jnp
from jax.experimental import pallas as pl
from jax.experimental.pallas import tpu as pltpu
from jax.experimental.pallas import tpu_sc as plsc

S, N, D = 1024, 8, 256
M = S * N
K = 8192
BM = 256


def _vq_block_kernel(x_ref, cb_ref, xsq_ref, csq_ref,
                     loss_ref, oh_ref, idx_ref):
    i = pl.program_id(0)
    x = x_ref[...]
    cb = cb_ref[...]
    mm = jax.lax.dot_general(x, cb, (((1,), (1,)), ((), ())),
                             preferred_element_type=jnp.float32)
    d = (xsq_ref[...] + csq_ref[...]) - 2.0 * mm
    dmin = jnp.min(d, axis=1, keepdims=True)
    kio = jax.lax.broadcasted_iota(jnp.int32, d.shape, 1)
    idx = jnp.min(jnp.where(d == dmin, kio, K), axis=1, keepdims=True)
    idx_ref[...] = idx
    oh_ref[...] = (kio == idx).astype(jnp.float32)

    @pl.when(i == 0)
    def _init():
        loss_ref[...] = jnp.zeros_like(loss_ref)

    loss_ref[...] += jnp.sum(dmin).reshape(1, 1)


_SC_INFO = plsc.get_sparse_core_info()
_NW = _SC_INFO.num_cores * _SC_INFO.num_subcores
_BPW = M // _NW


def _sc_gather_body(table_hbm, idx_hbm, out_hbm, idx_v, rows_v, sem):
    wid = (jax.lax.axis_index("s") * _SC_INFO.num_cores
           + jax.lax.axis_index("c"))
    base = wid * _BPW
    pltpu.sync_copy(idx_hbm.at[pl.ds(base, _BPW)], idx_v)
    pltpu.async_copy(table_hbm.at[idx_v], rows_v, sem).wait()
    pltpu.sync_copy(rows_v, out_hbm.at[pl.ds(base, _BPW)])


def _make_sc_gather():
    return functools.partial(
        pl.kernel,
        mesh=plsc.VectorSubcoreMesh(core_axis_name="c", subcore_axis_name="s"),
        out_type=jax.ShapeDtypeStruct((M, D), jnp.float32),
        scratch_types=[
            pltpu.VMEM((_BPW,), jnp.int32),
            pltpu.VMEM((_BPW, D), jnp.float32),
            pltpu.SemaphoreType.DMA,
        ],
    )(_sc_gather_body)


_sc_gather = _make_sc_gather()


@jax.jit
def kernel(inputs, codebook):
    flat = inputs.reshape(-1, D)
    xsq = jnp.sum(flat ** 2, axis=1, keepdims=True)
    csq = jnp.sum(codebook ** 2, axis=1)[None, :]

    grid = (M // BM,)
    loss_acc, oh, idx = pl.pallas_call(
        _vq_block_kernel,
        grid=grid,
        in_specs=[
            pl.BlockSpec((BM, D), lambda i: (i, 0)),
            pl.BlockSpec((K, D), lambda i: (0, 0)),
            pl.BlockSpec((BM, 1), lambda i: (i, 0)),
            pl.BlockSpec((1, K), lambda i: (0, 0)),
        ],
        out_specs=[
            pl.BlockSpec((1, 1), lambda i: (0, 0)),
            pl.BlockSpec((BM, K), lambda i: (i, 0)),
            pl.BlockSpec((BM, 1), lambda i: (i, 0)),
        ],
        out_shape=[
            jax.ShapeDtypeStruct((1, 1), jnp.float32),
            jax.ShapeDtypeStruct((M, K), jnp.float32),
            jax.ShapeDtypeStruct((M, 1), jnp.int32),
        ],
    )(flat, codebook, xsq, csq)

    q = _sc_gather(codebook, idx.reshape(M))

    loss = loss_acc[0, 0] * (0.25 / (M * D))
    quantized_st = q.reshape(S, N, D)
    encodings_flat = oh.reshape(S, N, K)
    return (loss, quantized_st, encodings_flat, idx)
